# Initial kernel scaffold; baseline (speedup 1.0000x reference)
import jax
import jax.numpy as jnp
from jax.experimental import pallas as pl


def kernel(player_loc, player_conf, player_loc_t, player_conf_t):
    raise NotImplementedError("write your pallas kernel here")


# speedup vs baseline: 8.9149x; 8.9149x over previous
import jax
import jax.numpy as jnp
from jax.experimental import pallas as pl
from jax.experimental.pallas import tpu as pltpu

_RATIO = 3


def _body(c0_ref, c1_ref, lab_ref, loc_ref, loct_ref, m4_ref,
          out_l_ref, out_c_ref, acc_ref, t_ref):
    b = pl.program_id(0)
    nb = pl.num_programs(0)

    @pl.when(b == 0)
    def _init():
        acc_ref[0] = 0.0
        acc_ref[1] = 0.0
        acc_ref[2] = 0.0

    c0 = c0_ref[0]
    c1 = c1_ref[0]
    lab = lab_ref[0]
    x = c1 - c0
    sp = jnp.maximum(x, 0.0) + jnp.log1p(jnp.exp(-jnp.abs(x)))
    pos = lab > 0
    neg = lab == 0
    np_i = jnp.sum(pos.astype(jnp.int32))
    nneg = jnp.sum(neg.astype(jnp.int32))
    np_cl = jnp.maximum(np_i, 1)
    k = jnp.minimum(np_cl * _RATIO, nneg)
    ce_pos = jnp.sum(jnp.where(pos, sp - x, 0.0))
    s_all = jnp.sum(jnp.where(neg, sp, 0.0))

    t_ref[0] = s_all

    @pl.when(jnp.logical_and(k < nneg, k > 0))
    def _search():
        u = jnp.where(neg, jax.lax.bitcast_convert_type(sp, jnp.int32),
                      jnp.int32(-1))
        p = jnp.int32(0)
        for i in range(30, -1, -1):
            cand = p | jnp.int32(1 << i)
            cnt = jnp.sum((u >= cand).astype(jnp.int32))
            p = jnp.where(cnt >= k, cand, p)
        gt = u > p
        cnt_gt = jnp.sum(gt.astype(jnp.int32))
        sum_gt = jnp.sum(jnp.where(gt, sp, 0.0))
        tval = jax.lax.bitcast_convert_type(p, jnp.float32)
        t_ref[0] = sum_gt + (k - cnt_gt).astype(jnp.float32) * tval

    d = loc_ref[0] - loct_ref[0]
    ad = jnp.abs(d)
    sl1 = jnp.where(ad < 1.0, 0.5 * d * d, ad - 0.5)
    ll = jnp.sum(jnp.where(m4_ref[0] != 0, sl1, 0.0))

    acc_ref[0] = acc_ref[0] + ll
    acc_ref[1] = acc_ref[1] + (ce_pos + t_ref[0])
    acc_ref[2] = acc_ref[2] + np_cl.astype(jnp.float32)

    @pl.when(b == nb - 1)
    def _fin():
        npf = acc_ref[2]
        out_l_ref[0, 0] = acc_ref[0] / npf
        out_c_ref[0, 0] = acc_ref[1] / npf


def kernel(player_loc, player_conf, player_loc_t, player_conf_t):
    B, P = player_conf_t.shape
    rows_c = -(-P // 128)
    ppad = rows_c * 128 - P
    rows_l = (P * 4) // 128

    c0 = jnp.pad(player_conf[:, :, 0], ((0, 0), (0, ppad)))
    c1 = jnp.pad(player_conf[:, :, 1], ((0, 0), (0, ppad)))
    labp = jnp.pad(player_conf_t, ((0, 0), (0, ppad)), constant_values=-1)
    c0 = c0.reshape(B, rows_c, 128)
    c1 = c1.reshape(B, rows_c, 128)
    labp = labp.reshape(B, rows_c, 128)
    locv = player_loc.reshape(B, rows_l, 128)
    loctv = player_loc_t.reshape(B, rows_l, 128)
    m4 = jnp.repeat((player_conf_t > 0).astype(jnp.int8), 4,
                    axis=1).reshape(B, rows_l, 128)

    row = lambda i: (i, 0, 0)
    out_l, out_c = pl.pallas_call(
        _body,
        grid=(B,),
        in_specs=[
            pl.BlockSpec((1, rows_c, 128), row),
            pl.BlockSpec((1, rows_c, 128), row),
            pl.BlockSpec((1, rows_c, 128), row),
            pl.BlockSpec((1, rows_l, 128), row),
            pl.BlockSpec((1, rows_l, 128), row),
            pl.BlockSpec((1, rows_l, 128), row),
        ],
        out_specs=[
            pl.BlockSpec(memory_space=pltpu.SMEM),
            pl.BlockSpec(memory_space=pltpu.SMEM),
        ],
        out_shape=[
            jax.ShapeDtypeStruct((1, 1), jnp.float32),
            jax.ShapeDtypeStruct((1, 1), jnp.float32),
        ],
        scratch_shapes=[
            pltpu.SMEM((3,), jnp.float32),
            pltpu.SMEM((1,), jnp.float32),
        ],
        compiler_params=pltpu.CompilerParams(
            dimension_semantics=("arbitrary",),
        ),
    )(c0, c1, labp, locv, loctv, m4)
    return (out_l[0, 0], out_c[0, 0])


# speedup vs baseline: 11.7087x; 1.3134x over previous
import jax
import jax.numpy as jnp
from jax.experimental import pallas as pl
from jax.experimental.pallas import tpu as pltpu

_RATIO = 3


def _softplus(x):
    return jnp.maximum(x, 0.0) + jnp.log1p(jnp.exp(-jnp.abs(x)))


def _body(c0_ref, c1_ref, lab_ref, loc_ref, loct_ref, m4_ref,
          out_l_ref, out_c_ref, acc_ref):
    step = pl.program_id(0)
    nsteps = pl.num_programs(0)
    C = c0_ref.shape[0]

    @pl.when(step == 0)
    def _init():
        acc_ref[0] = 0.0
        acc_ref[1] = 0.0
        acc_ref[2] = 0.0

    c0 = c0_ref[...]
    c1 = c1_ref[...]
    lab = lab_ref[...].astype(jnp.int32)
    x = c1 - c0
    sp = _softplus(x)
    pos = lab > 0
    neg = lab == 0
    np_v = jnp.sum(pos.astype(jnp.int32), axis=(1, 2), keepdims=True)
    nneg_v = jnp.sum(neg.astype(jnp.int32), axis=(1, 2), keepdims=True)
    np_cl = jnp.maximum(np_v, 1)
    k_v = jnp.minimum(np_cl * _RATIO, nneg_v)

    ce_pos = jnp.sum(jnp.where(pos, sp - x, 0.0))
    s_all = jnp.sum(jnp.where(neg, sp, 0.0))
    ll = _loc_loss(loc_ref, loct_ref, m4_ref)

    flags = jnp.logical_and(k_v < nneg_v, k_v > 0)
    nflag = jnp.sum(flags.astype(jnp.int32))

    acc_ref[0] = acc_ref[0] + ll
    acc_ref[1] = acc_ref[1] + ce_pos + s_all
    acc_ref[2] = acc_ref[2] + jnp.sum(np_cl).astype(jnp.float32)

    @pl.when(nflag > 0)
    def _slow():
        def row(r, tot):
            labr = lab_ref[r].astype(jnp.int32)
            c0r = c0_ref[r]
            c1r = c1_ref[r]
            xr = c1r - c0r
            spr = _softplus(xr)
            negr = labr == 0
            np_r = jnp.sum((labr > 0).astype(jnp.int32))
            nneg_r = jnp.sum(negr.astype(jnp.int32))
            k_r = jnp.minimum(jnp.maximum(np_r, 1) * _RATIO, nneg_r)
            flag_r = jnp.logical_and(k_r < nneg_r, k_r > 0)
            s_all_r = jnp.sum(jnp.where(negr, spr, 0.0))
            u = jnp.where(negr,
                          jax.lax.bitcast_convert_type(spr, jnp.int32),
                          jnp.int32(-1))
            p = jnp.int32(0)
            for i in range(30, -1, -1):
                cand = p | jnp.int32(1 << i)
                cnt = jnp.sum((u >= cand).astype(jnp.int32))
                p = jnp.where(cnt >= k_r, cand, p)
            gt = u > p
            cnt_gt = jnp.sum(gt.astype(jnp.int32))
            sum_gt = jnp.sum(jnp.where(gt, spr, 0.0))
            tval = jax.lax.bitcast_convert_type(p, jnp.float32)
            topk = sum_gt + (k_r - cnt_gt).astype(jnp.float32) * tval
            return tot + jnp.where(flag_r, topk - s_all_r, 0.0)

        fix = jax.lax.fori_loop(0, C, row, 0.0)
        acc_ref[1] = acc_ref[1] + fix

    @pl.when(step == nsteps - 1)
    def _fin():
        npf = acc_ref[2]
        out_l_ref[0, 0] = acc_ref[0] / npf
        out_c_ref[0, 0] = acc_ref[1] / npf


def _loc_loss(loc_ref, loct_ref, m4_ref):
    d = loc_ref[...] - loct_ref[...]
    ad = jnp.abs(d)
    sl1 = jnp.where(ad < 1.0, 0.5 * d * d, ad - 0.5)
    return jnp.sum(jnp.where(m4_ref[...] != 0, sl1, 0.0))


def kernel(player_loc, player_conf, player_loc_t, player_conf_t):
    B, P = player_conf_t.shape
    rows_c = -(-P // 128)
    ppad = rows_c * 128 - P
    rows_l = (P * 4) // 128
    C = 8 if B % 8 == 0 else 1

    c0 = jnp.pad(player_conf[:, :, 0], ((0, 0), (0, ppad)))
    c1 = jnp.pad(player_conf[:, :, 1], ((0, 0), (0, ppad)))
    labp = jnp.pad(player_conf_t.astype(jnp.int8), ((0, 0), (0, ppad)),
                   constant_values=-1)
    c0 = c0.reshape(B, rows_c, 128)
    c1 = c1.reshape(B, rows_c, 128)
    labp = labp.reshape(B, rows_c, 128)
    locv = player_loc.reshape(B, rows_l, 128)
    loctv = player_loc_t.reshape(B, rows_l, 128)
    m4 = jnp.repeat((player_conf_t > 0).astype(jnp.int8), 4,
                    axis=1).reshape(B, rows_l, 128)

    row = lambda i: (i, 0, 0)
    out_l, out_c = pl.pallas_call(
        _body,
        grid=(B // C,),
        in_specs=[
            pl.BlockSpec((C, rows_c, 128), row),
            pl.BlockSpec((C, rows_c, 128), row),
            pl.BlockSpec((C, rows_c, 128), row),
            pl.BlockSpec((C, rows_l, 128), row),
            pl.BlockSpec((C, rows_l, 128), row),
            pl.BlockSpec((C, rows_l, 128), row),
        ],
        out_specs=[
            pl.BlockSpec(memory_space=pltpu.SMEM),
            pl.BlockSpec(memory_space=pltpu.SMEM),
        ],
        out_shape=[
            jax.ShapeDtypeStruct((1, 1), jnp.float32),
            jax.ShapeDtypeStruct((1, 1), jnp.float32),
        ],
        scratch_shapes=[
            pltpu.SMEM((3,), jnp.float32),
        ],
        compiler_params=pltpu.CompilerParams(
            dimension_semantics=("arbitrary",),
        ),
    )(c0, c1, labp, locv, loctv, m4)
    return (out_l[0, 0], out_c[0, 0])
